# baseline (speedup 1.0000x reference)
import functools

import jax
import jax.numpy as jnp
import numpy as np
from jax import lax
from jax.experimental import pallas as pl
from jax.experimental.pallas import tpu as pltpu
from jax.experimental.pallas import tpu_sc as plsc

_NUM_LEVELS = 16
_BASE_RES = 16
_LOG2 = 21
_F = 2
_B = 4
_N = 65536

_P1 = int(np.uint32(2654435761).astype(np.int32))
_P2 = 805459861
_MASK = 2**_LOG2 - 1

_NC = 2
_NS = 16
_NW = _NC * _NS
_PTS = _B * _N
_PER_W = _PTS // _NW
_C = 1024
_CHUNKS = _PER_W // _C
_GROUPS = _C // 16
_WPB = _N // _PER_W


def _tc_normalize(c_ref, o_ref):
    c = c_ref[...]
    cmin = jnp.min(c, axis=2, keepdims=True)
    cmax = jnp.max(c, axis=2, keepdims=True)
    o_ref[...] = (c - cmin) / (cmax - cmin + 1e-6)


def _sc_body(cn_hbm, tbl_hbm, out_hbm, cn_v, frac_v, idx_v, rows_v, outb_v, sem):
    wid = lax.axis_index("s") * _NC + lax.axis_index("c")
    b = wid // _WPB
    nbase = (wid % _WPB) * _PER_W
    iota = lax.broadcasted_iota(jnp.int32, (16,), 0)
    zeros16 = jnp.zeros((16,), jnp.int32)
    ones16 = jnp.full((16,), 1, jnp.int32)

    def chunk_body(ci, carry):
        n0 = nbase + ci * _C
        pltpu.sync_copy(cn_hbm.at[b, :, pl.ds(n0, _C)], cn_v)

        def level_body(l, carry2):
            scale = lax.shift_left(_BASE_RES, l).astype(jnp.float32)
            lvl_off = lax.shift_left(l, _LOG2)

            def hash_body(g, carry3):
                p0 = g * 16
                xs = cn_v[0, pl.ds(p0, 16)] * scale
                ys = cn_v[1, pl.ds(p0, 16)] * scale
                zs = cn_v[2, pl.ds(p0, 16)] * scale
                xi = xs.astype(jnp.int32)
                yi = ys.astype(jnp.int32)
                zi = zs.astype(jnp.int32)
                frac_v[0, pl.ds(p0, 16)] = xs - xi.astype(jnp.float32)
                frac_v[1, pl.ds(p0, 16)] = ys - yi.astype(jnp.float32)
                frac_v[2, pl.ds(p0, 16)] = zs - zi.astype(jnp.float32)
                ax0 = xi
                ax1 = xi + 1
                by0 = yi * _P1
                by1 = by0 + _P1
                cz0 = zi * _P2
                cz1 = cz0 + _P2
                for k in range(8):
                    a = ax1 if (k >> 2) & 1 else ax0
                    bb = by1 if (k >> 1) & 1 else by0
                    cc = cz1 if k & 1 else cz0
                    h = ((a + bb + cc) & _MASK) + lvl_off
                    idx_v[pl.ds(k * _C + p0, 16)] = h
                return carry3

            lax.fori_loop(0, _GROUPS, hash_body, 0)

            pltpu.async_copy(tbl_hbm.at[idx_v], rows_v, sem).wait()

            def blend_body(g, carry3):
                p0 = g * 16
                fx = frac_v[0, pl.ds(p0, 16)]
                fy = frac_v[1, pl.ds(p0, 16)]
                fz = frac_v[2, pl.ds(p0, 16)]
                gx = 1.0 - fx
                gy = 1.0 - fy
                gz = 1.0 - fz
                t00 = gy * gz
                t01 = gy * fz
                t10 = fy * gz
                t11 = fy * fz
                ws = (gx * t00, gx * t01, gx * t10, gx * t11,
                      fx * t00, fx * t01, fx * t10, fx * t11)
                acc0 = jnp.zeros((16,), jnp.float32)
                acc1 = jnp.zeros((16,), jnp.float32)
                for k in range(8):
                    rowid = k * _C + p0 + iota
                    r0 = plsc.load_gather(rows_v, [rowid, zeros16])
                    r1 = plsc.load_gather(rows_v, [rowid, ones16])
                    acc0 = acc0 + ws[k] * r0
                    acc1 = acc1 + ws[k] * r1
                outb_v[0, pl.ds(p0, 16)] = acc0
                outb_v[1, pl.ds(p0, 16)] = acc1
                return carry3

            lax.fori_loop(0, _GROUPS, blend_body, 0)

            pltpu.sync_copy(outb_v, out_hbm.at[b, pl.ds(2 * l, 2), pl.ds(n0, _C)])
            return carry2

        lax.fori_loop(0, _NUM_LEVELS, level_body, 0)
        return carry

    lax.fori_loop(0, _CHUNKS, chunk_body, 0)


def kernel(coords, tables):
    cnorm = pl.pallas_call(
        _tc_normalize,
        out_shape=jax.ShapeDtypeStruct((_B, 3, _N), jnp.float32),
    )(coords)

    tbl = tables.reshape(_NUM_LEVELS * 2**_LOG2, _F)
    mesh = plsc.VectorSubcoreMesh(core_axis_name="c", subcore_axis_name="s")
    sc = functools.partial(
        pl.kernel,
        mesh=mesh,
        compiler_params=pltpu.CompilerParams(
            needs_layout_passes=False, use_tc_tiling_on_sc=False
        ),
        out_type=jax.ShapeDtypeStruct((_B, 2 * _NUM_LEVELS, _N), jnp.float32),
        scratch_types=[
            pltpu.VMEM((3, _C), jnp.float32),
            pltpu.VMEM((3, _C), jnp.float32),
            pltpu.VMEM((8 * _C,), jnp.int32),
            pltpu.VMEM((8 * _C, _F), jnp.float32),
            pltpu.VMEM((2, _C), jnp.float32),
            pltpu.SemaphoreType.DMA,
        ],
    )(_sc_body)
    return sc(cnorm, tbl)


# speedup vs baseline: 21.4123x; 21.4123x over previous
import functools

import jax
import jax.numpy as jnp
import numpy as np
from jax import lax
from jax.experimental import pallas as pl
from jax.experimental.pallas import tpu as pltpu
from jax.experimental.pallas import tpu_sc as plsc

_NUM_LEVELS = 16
_BASE_RES = 16
_LOG2 = 21
_F = 2
_B = 4
_N = 65536

_P1 = int(np.uint32(2654435761).astype(np.int32))
_P2 = 805459861
_MASK = 2**_LOG2 - 1

_NC = 2
_NS = 16
_NW = _NC * _NS
_PTS = _B * _N
_PER_W = _PTS // _NW
_C = 1024
_CHUNKS = _PER_W // _C
_GROUPS = _C // 16
_WPB = _N // _PER_W

_BT = 2048


def _tc_normalize(c_ref, o_ref):
    c = c_ref[...]
    cmin = jnp.min(c, axis=2, keepdims=True)
    cmax = jnp.max(c, axis=2, keepdims=True)
    o_ref[...] = (c - cmin) / (cmax - cmin + 1e-6)


def _tc_interleave(t_ref, o_ref):
    x = t_ref[0]
    x2 = jnp.concatenate([x[:, 0, :], x[:, 1, :]], axis=1)
    row = lax.broadcasted_iota(jnp.int32, (256, 128), 0)
    lane = lax.broadcasted_iota(jnp.int32, (256, 128), 1)
    cond_e = ((row < 64) & (lane == 2 * row)) | (
        (row >= 128) & (row < 192) & (lane == 2 * (row - 128) + 1))
    cond_o = ((row >= 64) & (row < 128) & (lane == 2 * (row - 64))) | (
        (row >= 192) & (lane == 2 * (row - 192) + 1))
    m_e = jnp.where(cond_e, 1.0, 0.0).astype(jnp.float32)
    m_o = jnp.where(cond_o, 1.0, 0.0).astype(jnp.float32)
    o_e = jax.lax.dot(x2, m_e, precision=jax.lax.Precision.HIGHEST,
                      preferred_element_type=jnp.float32)
    o_o = jax.lax.dot(x2, m_o, precision=jax.lax.Precision.HIGHEST,
                      preferred_element_type=jnp.float32)
    y = jnp.concatenate([o_e[:, None, :], o_o[:, None, :]], axis=1)
    o_ref[...] = y.reshape(2 * _BT, 128)


def _sc_body(cn_hbm, tbl_hbm, out_hbm, cn_v, frac_v, idx_v, pos_v, rows_v,
             outb_v, sem):
    wid = lax.axis_index("s") * _NC + lax.axis_index("c")
    b = wid // _WPB
    nbase = (wid % _WPB) * _PER_W
    iota = lax.broadcasted_iota(jnp.int32, (16,), 0)

    def chunk_body(ci, carry):
        n0 = nbase + ci * _C
        pltpu.sync_copy(cn_hbm.at[b, :, pl.ds(n0, _C)], cn_v)

        def level_body(l, carry2):
            scale = lax.shift_left(_BASE_RES, l).astype(jnp.float32)
            lvl_off = lax.shift_left(l, _LOG2 - 2)

            def hash_body(g, carry3):
                p0 = g * 16
                xs = cn_v[0, pl.ds(p0, 16)] * scale
                ys = cn_v[1, pl.ds(p0, 16)] * scale
                zs = cn_v[2, pl.ds(p0, 16)] * scale
                xi = xs.astype(jnp.int32)
                yi = ys.astype(jnp.int32)
                zi = zs.astype(jnp.int32)
                frac_v[0, pl.ds(p0, 16)] = xs - xi.astype(jnp.float32)
                frac_v[1, pl.ds(p0, 16)] = ys - yi.astype(jnp.float32)
                frac_v[2, pl.ds(p0, 16)] = zs - zi.astype(jnp.float32)
                ax0 = xi
                ax1 = xi + 1
                by0 = yi * _P1
                by1 = by0 + _P1
                cz0 = zi * _P2
                cz1 = cz0 + _P2
                for k in range(8):
                    a = ax1 if (k >> 2) & 1 else ax0
                    bb = by1 if (k >> 1) & 1 else by0
                    cc = cz1 if k & 1 else cz0
                    h = (a + bb + cc) & _MASK
                    idx_v[pl.ds(k * _C + p0, 16)] = (
                        lax.shift_right_logical(h, 2) + lvl_off)
                    pos_v[pl.ds(k * _C + p0, 16)] = h & 3
                return carry3

            lax.fori_loop(0, _GROUPS, hash_body, 0)

            pltpu.async_copy(tbl_hbm.at[idx_v], rows_v, sem).wait()

            def blend_body(g, carry3):
                p0 = g * 16
                fx = frac_v[0, pl.ds(p0, 16)]
                fy = frac_v[1, pl.ds(p0, 16)]
                fz = frac_v[2, pl.ds(p0, 16)]
                gx = 1.0 - fx
                gy = 1.0 - fy
                gz = 1.0 - fz
                t00 = gy * gz
                t01 = gy * fz
                t10 = fy * gz
                t11 = fy * fz
                ws = (gx * t00, gx * t01, gx * t10, gx * t11,
                      fx * t00, fx * t01, fx * t10, fx * t11)
                acc0 = jnp.zeros((16,), jnp.float32)
                acc1 = jnp.zeros((16,), jnp.float32)
                for k in range(8):
                    rowid = k * _C + p0 + iota
                    pos2 = pos_v[pl.ds(k * _C + p0, 16)] * 2
                    r0 = plsc.load_gather(rows_v, [rowid, pos2])
                    r1 = plsc.load_gather(rows_v, [rowid, pos2 + 1])
                    acc0 = acc0 + ws[k] * r0
                    acc1 = acc1 + ws[k] * r1
                outb_v[0, pl.ds(p0, 16)] = acc0
                outb_v[1, pl.ds(p0, 16)] = acc1
                return carry3

            lax.fori_loop(0, _GROUPS, blend_body, 0)

            pltpu.sync_copy(outb_v, out_hbm.at[b, pl.ds(2 * l, 2), pl.ds(n0, _C)])
            return carry2

        lax.fori_loop(0, _NUM_LEVELS, level_body, 0)
        return carry

    lax.fori_loop(0, _CHUNKS, chunk_body, 0)


def kernel(coords, tables):
    cnorm = pl.pallas_call(
        _tc_normalize,
        out_shape=jax.ShapeDtypeStruct((_B, 3, _N), jnp.float32),
    )(coords)

    tv = tables.reshape(_NUM_LEVELS, 2**_LOG2 // 128, 128, _F).swapaxes(2, 3)
    ncell = (2**_LOG2 // 128) // _BT
    tbl128 = pl.pallas_call(
        _tc_interleave,
        grid=(_NUM_LEVELS, ncell),
        in_specs=[pl.BlockSpec((1, _BT, _F, 128), lambda l, c: (l, c, 0, 0))],
        out_specs=pl.BlockSpec((2 * _BT, 128), lambda l, c: (l * ncell + c, 0)),
        out_shape=jax.ShapeDtypeStruct(
            (_NUM_LEVELS * 2**_LOG2 * _F // 128, 128), jnp.float32),
    )(tv)
    tbl = tbl128.reshape(_NUM_LEVELS * 2**_LOG2 * _F // 8, 8)

    mesh = plsc.VectorSubcoreMesh(core_axis_name="c", subcore_axis_name="s")
    sc = functools.partial(
        pl.kernel,
        mesh=mesh,
        compiler_params=pltpu.CompilerParams(
            needs_layout_passes=False, use_tc_tiling_on_sc=False
        ),
        out_type=jax.ShapeDtypeStruct((_B, 2 * _NUM_LEVELS, _N), jnp.float32),
        scratch_types=[
            pltpu.VMEM((3, _C), jnp.float32),
            pltpu.VMEM((3, _C), jnp.float32),
            pltpu.VMEM((8 * _C,), jnp.int32),
            pltpu.VMEM((8 * _C,), jnp.int32),
            pltpu.VMEM((8 * _C, 8), jnp.float32),
            pltpu.VMEM((2, _C), jnp.float32),
            pltpu.SemaphoreType.DMA,
        ],
    )(_sc_body)
    return sc(cnorm, tbl)


# speedup vs baseline: 26.2695x; 1.2268x over previous
import functools

import jax
import jax.numpy as jnp
import numpy as np
from jax import lax
from jax.experimental import pallas as pl
from jax.experimental.pallas import tpu as pltpu
from jax.experimental.pallas import tpu_sc as plsc

_NUM_LEVELS = 16
_BASE_RES = 16
_LOG2 = 21
_F = 2
_B = 4
_N = 65536

_P1 = int(np.uint32(2654435761).astype(np.int32))
_P2 = 805459861
_MASK = 2**_LOG2 - 1

_NC = 2
_NS = 16
_NW = _NC * _NS
_PTS = _B * _N
_PER_W = _PTS // _NW
_C = 512
_CHUNKS = _PER_W // _C
_GROUPS = _C // 16
_WPB = _N // _PER_W

_BT = 2048


def _tc_normalize(c_ref, o_ref):
    c = c_ref[...]
    cmin = jnp.min(c, axis=2, keepdims=True)
    cmax = jnp.max(c, axis=2, keepdims=True)
    o_ref[...] = (c - cmin) / (cmax - cmin + 1e-6)


def _tc_interleave(t_ref, o_ref):
    x = t_ref[0]
    x2 = jnp.concatenate([x[:, 0, :], x[:, 1, :]], axis=1)
    row = lax.broadcasted_iota(jnp.int32, (256, 128), 0)
    lane = lax.broadcasted_iota(jnp.int32, (256, 128), 1)
    cond_e = ((row < 64) & (lane == 2 * row)) | (
        (row >= 128) & (row < 192) & (lane == 2 * (row - 128) + 1))
    cond_o = ((row >= 64) & (row < 128) & (lane == 2 * (row - 64))) | (
        (row >= 192) & (lane == 2 * (row - 192) + 1))
    m_e = jnp.where(cond_e, 1.0, 0.0).astype(jnp.float32)
    m_o = jnp.where(cond_o, 1.0, 0.0).astype(jnp.float32)
    o_e = jax.lax.dot(x2, m_e, precision=jax.lax.Precision.HIGHEST,
                      preferred_element_type=jnp.float32)
    o_o = jax.lax.dot(x2, m_o, precision=jax.lax.Precision.HIGHEST,
                      preferred_element_type=jnp.float32)
    y = jnp.concatenate([o_e[:, None, :], o_o[:, None, :]], axis=1)
    o_ref[...] = y.reshape(2 * _BT, 128)


def _sc_body(cn_hbm, tbl_hbm, out_hbm, cn_v,
             frac_a, frac_b, idx_a, idx_b, pos_a, pos_b, rows_a, rows_b,
             outb_a, outb_b, sem_a, sem_b, sem_oa, sem_ob):
    wid = lax.axis_index("s") * _NC + lax.axis_index("c")
    b = wid // _WPB
    nbase = (wid % _WPB) * _PER_W
    iota = lax.broadcasted_iota(jnp.int32, (16,), 0)
    bufs = ((frac_a, idx_a, pos_a, rows_a, sem_a),
            (frac_b, idx_b, pos_b, rows_b, sem_b))
    obufs = ((outb_a, sem_oa), (outb_b, sem_ob))

    def hash_level(l, frac_v, idx_v, pos_v):
        scale = float(_BASE_RES * 2.0 ** l)
        lvl_off = l << (_LOG2 - 2)

        def hash_body(g, carry):
            p0 = g * 16
            xs = cn_v[0, pl.ds(p0, 16)] * scale
            ys = cn_v[1, pl.ds(p0, 16)] * scale
            zs = cn_v[2, pl.ds(p0, 16)] * scale
            xi = xs.astype(jnp.int32)
            yi = ys.astype(jnp.int32)
            zi = zs.astype(jnp.int32)
            frac_v[0, pl.ds(p0, 16)] = xs - xi.astype(jnp.float32)
            frac_v[1, pl.ds(p0, 16)] = ys - yi.astype(jnp.float32)
            frac_v[2, pl.ds(p0, 16)] = zs - zi.astype(jnp.float32)
            ax0 = xi
            ax1 = xi + 1
            by0 = yi * _P1
            by1 = by0 + _P1
            cz0 = zi * _P2
            cz1 = cz0 + _P2
            for k in range(8):
                a = ax1 if (k >> 2) & 1 else ax0
                bb = by1 if (k >> 1) & 1 else by0
                cc = cz1 if k & 1 else cz0
                h = (a + bb + cc) & _MASK
                idx_v[pl.ds(k * _C + p0, 16)] = (
                    lax.shift_right_logical(h, 2) + lvl_off)
                pos_v[pl.ds(k * _C + p0, 16)] = h & 3
            return carry

        lax.fori_loop(0, _GROUPS, hash_body, 0)

    def blend_level(frac_v, pos_v, rows_v, outb_v):
        def blend_body(g, carry):
            p0 = g * 16
            fx = frac_v[0, pl.ds(p0, 16)]
            fy = frac_v[1, pl.ds(p0, 16)]
            fz = frac_v[2, pl.ds(p0, 16)]
            gx = 1.0 - fx
            gy = 1.0 - fy
            gz = 1.0 - fz
            t00 = gy * gz
            t01 = gy * fz
            t10 = fy * gz
            t11 = fy * fz
            ws = (gx * t00, gx * t01, gx * t10, gx * t11,
                  fx * t00, fx * t01, fx * t10, fx * t11)
            acc0 = jnp.zeros((16,), jnp.float32)
            acc1 = jnp.zeros((16,), jnp.float32)
            for k in range(8):
                rowid = k * _C + p0 + iota
                pos2 = pos_v[pl.ds(k * _C + p0, 16)] * 2
                r0 = plsc.load_gather(rows_v, [rowid, pos2])
                r1 = plsc.load_gather(rows_v, [rowid, pos2 + 1])
                acc0 = acc0 + ws[k] * r0
                acc1 = acc1 + ws[k] * r1
            outb_v[0, pl.ds(p0, 16)] = acc0
            outb_v[1, pl.ds(p0, 16)] = acc1
            return carry

        lax.fori_loop(0, _GROUPS, blend_body, 0)

    def chunk_body(ci, carry):
        n0 = nbase + ci * _C
        pltpu.sync_copy(cn_hbm.at[b, :, pl.ds(n0, _C)], cn_v)

        frac_v, idx_v, pos_v, rows_v, sem = bufs[0]
        hash_level(0, frac_v, idx_v, pos_v)
        gh = {0: pltpu.async_copy(tbl_hbm.at[idx_v], rows_v, sem)}
        oh = {0: None, 1: None}
        for l in range(_NUM_LEVELS):
            p = l & 1
            frac_v, idx_v, pos_v, rows_v, sem = bufs[p]
            if l + 1 < _NUM_LEVELS:
                nfrac, nidx, npos, nrows, nsem = bufs[1 - p]
                hash_level(l + 1, nfrac, nidx, npos)
                gh[1 - p] = pltpu.async_copy(tbl_hbm.at[nidx], nrows, nsem)
            gh[p].wait()
            outb_v, sem_o = obufs[p]
            if oh[p] is not None:
                oh[p].wait()
            blend_level(frac_v, pos_v, rows_v, outb_v)
            oh[p] = pltpu.async_copy(
                outb_v, out_hbm.at[b, pl.ds(2 * l, 2), pl.ds(n0, _C)], sem_o)
        oh[0].wait()
        oh[1].wait()
        return carry

    lax.fori_loop(0, _CHUNKS, chunk_body, 0)


def kernel(coords, tables):
    cnorm = pl.pallas_call(
        _tc_normalize,
        out_shape=jax.ShapeDtypeStruct((_B, 3, _N), jnp.float32),
    )(coords)

    tv = tables.reshape(_NUM_LEVELS, 2**_LOG2 // 128, 128, _F).swapaxes(2, 3)
    ncell = (2**_LOG2 // 128) // _BT
    tbl128 = pl.pallas_call(
        _tc_interleave,
        grid=(_NUM_LEVELS, ncell),
        in_specs=[pl.BlockSpec((1, _BT, _F, 128), lambda l, c: (l, c, 0, 0))],
        out_specs=pl.BlockSpec((2 * _BT, 128), lambda l, c: (l * ncell + c, 0)),
        out_shape=jax.ShapeDtypeStruct(
            (_NUM_LEVELS * 2**_LOG2 * _F // 128, 128), jnp.float32),
    )(tv)
    tbl = tbl128.reshape(_NUM_LEVELS * 2**_LOG2 * _F // 8, 8)

    mesh = plsc.VectorSubcoreMesh(core_axis_name="c", subcore_axis_name="s")
    sc = functools.partial(
        pl.kernel,
        mesh=mesh,
        compiler_params=pltpu.CompilerParams(
            needs_layout_passes=False, use_tc_tiling_on_sc=False
        ),
        out_type=jax.ShapeDtypeStruct((_B, 2 * _NUM_LEVELS, _N), jnp.float32),
        scratch_types=[
            pltpu.VMEM((3, _C), jnp.float32),
            pltpu.VMEM((3, _C), jnp.float32),
            pltpu.VMEM((3, _C), jnp.float32),
            pltpu.VMEM((8 * _C,), jnp.int32),
            pltpu.VMEM((8 * _C,), jnp.int32),
            pltpu.VMEM((8 * _C,), jnp.int32),
            pltpu.VMEM((8 * _C,), jnp.int32),
            pltpu.VMEM((8 * _C, 8), jnp.float32),
            pltpu.VMEM((8 * _C, 8), jnp.float32),
            pltpu.VMEM((2, _C), jnp.float32),
            pltpu.VMEM((2, _C), jnp.float32),
            pltpu.SemaphoreType.DMA,
            pltpu.SemaphoreType.DMA,
            pltpu.SemaphoreType.DMA,
            pltpu.SemaphoreType.DMA,
        ],
    )(_sc_body)
    return sc(cnorm, tbl)


# speedup vs baseline: 26.2800x; 1.0004x over previous
import functools

import jax
import jax.numpy as jnp
import numpy as np
from jax import lax
from jax.experimental import pallas as pl
from jax.experimental.pallas import tpu as pltpu
from jax.experimental.pallas import tpu_sc as plsc

_NUM_LEVELS = 16
_BASE_RES = 16
_LOG2 = 21
_F = 2
_B = 4
_N = 65536

_P1 = int(np.uint32(2654435761).astype(np.int32))
_P2 = 805459861
_MASK = 2**_LOG2 - 1

_NC = 2
_NS = 16
_NW = _NC * _NS
_PTS = _B * _N
_PER_W = _PTS // _NW
_C = 512
_CHUNKS = _PER_W // _C
_GROUPS = _C // 16
_WPB = _N // _PER_W

_BT = 2048


def _tc_normalize(c_ref, o_ref):
    c = c_ref[...]
    cmin = jnp.min(c, axis=2, keepdims=True)
    cmax = jnp.max(c, axis=2, keepdims=True)
    o_ref[...] = (c - cmin) / (cmax - cmin + 1e-6)


def _tc_interleave(t_ref, o_ref):
    x = t_ref[0]
    x2 = jnp.concatenate([x[:, 0, :], x[:, 1, :]], axis=1)
    row = lax.broadcasted_iota(jnp.int32, (256, 128), 0)
    lane = lax.broadcasted_iota(jnp.int32, (256, 128), 1)
    cond_e = ((row < 64) & (lane == 2 * row)) | (
        (row >= 128) & (row < 192) & (lane == 2 * (row - 128) + 1))
    cond_o = ((row >= 64) & (row < 128) & (lane == 2 * (row - 64))) | (
        (row >= 192) & (lane == 2 * (row - 192) + 1))
    m_e = jnp.where(cond_e, 1.0, 0.0).astype(jnp.float32)
    m_o = jnp.where(cond_o, 1.0, 0.0).astype(jnp.float32)
    o_e = jax.lax.dot(x2, m_e, precision=jax.lax.Precision.HIGHEST,
                      preferred_element_type=jnp.float32)
    o_o = jax.lax.dot(x2, m_o, precision=jax.lax.Precision.HIGHEST,
                      preferred_element_type=jnp.float32)
    y = jnp.concatenate([o_e[:, None, :], o_o[:, None, :]], axis=1)
    o_ref[...] = y.reshape(2 * _BT, 128)


def _sc_body(cn_hbm, tbl_hbm, out_hbm, cn_v,
             frac_a, frac_b, idx_a, idx_b, pos_a, pos_b, rows_a, rows_b,
             outb_a, outb_b, sem_a, sem_b, sem_oa, sem_ob):
    wid = lax.axis_index("s") * _NC + lax.axis_index("c")
    b = wid // _WPB
    nbase = (wid % _WPB) * _PER_W
    iota = lax.broadcasted_iota(jnp.int32, (16,), 0)
    bufs = ((frac_a, idx_a, pos_a, rows_a, sem_a),
            (frac_b, idx_b, pos_b, rows_b, sem_b))
    obufs = ((outb_a, sem_oa), (outb_b, sem_ob))

    def hash_level(l, frac_v, idx_v, pos_v):
        scale = float(_BASE_RES * 2.0 ** l)
        lvl_off = l << (_LOG2 - 2)

        def hash_body(g, carry):
            p0 = g * 16
            xs = cn_v[0, pl.ds(p0, 16)] * scale
            ys = cn_v[1, pl.ds(p0, 16)] * scale
            zs = cn_v[2, pl.ds(p0, 16)] * scale
            xi = xs.astype(jnp.int32)
            yi = ys.astype(jnp.int32)
            zi = zs.astype(jnp.int32)
            frac_v[0, pl.ds(p0, 16)] = xs - xi.astype(jnp.float32)
            frac_v[1, pl.ds(p0, 16)] = ys - yi.astype(jnp.float32)
            frac_v[2, pl.ds(p0, 16)] = zs - zi.astype(jnp.float32)
            ax0 = xi
            ax1 = xi + 1
            by0 = yi * _P1
            by1 = by0 + _P1
            cz0 = zi * _P2
            cz1 = cz0 + _P2
            for k in range(8):
                a = ax1 if (k >> 2) & 1 else ax0
                bb = by1 if (k >> 1) & 1 else by0
                cc = cz1 if k & 1 else cz0
                h = (a + bb + cc) & _MASK
                idx_v[pl.ds(k * _C + p0, 16)] = (
                    lax.shift_right_logical(h, 2) + lvl_off)
                pos_v[pl.ds(k * _C + p0, 16)] = (h & 3) * 2
            return carry

        lax.fori_loop(0, _GROUPS, hash_body, 0)

    def blend_level(frac_v, pos_v, rows_v, outb_v):
        def blend_body(g, carry):
            p0 = g * 16
            fx = frac_v[0, pl.ds(p0, 16)]
            fy = frac_v[1, pl.ds(p0, 16)]
            fz = frac_v[2, pl.ds(p0, 16)]
            gx = 1.0 - fx
            gy = 1.0 - fy
            gz = 1.0 - fz
            t00 = gy * gz
            t01 = gy * fz
            t10 = fy * gz
            t11 = fy * fz
            ws = (gx * t00, gx * t01, gx * t10, gx * t11,
                  fx * t00, fx * t01, fx * t10, fx * t11)
            acc0 = jnp.zeros((16,), jnp.float32)
            acc1 = jnp.zeros((16,), jnp.float32)
            for k in range(8):
                rowid = k * _C + p0 + iota
                pos2 = pos_v[pl.ds(k * _C + p0, 16)]
                r0 = plsc.load_gather(rows_v, [rowid, pos2])
                r1 = plsc.load_gather(rows_v, [rowid, pos2 + 1])
                acc0 = acc0 + ws[k] * r0
                acc1 = acc1 + ws[k] * r1
            outb_v[0, pl.ds(p0, 16)] = acc0
            outb_v[1, pl.ds(p0, 16)] = acc1
            return carry

        lax.fori_loop(0, _GROUPS, blend_body, 0)

    def chunk_body(ci, carry):
        n0 = nbase + ci * _C
        pltpu.sync_copy(cn_hbm.at[b, :, pl.ds(n0, _C)], cn_v)

        frac_v, idx_v, pos_v, rows_v, sem = bufs[0]
        hash_level(0, frac_v, idx_v, pos_v)
        gh = {0: pltpu.async_copy(tbl_hbm.at[idx_v], rows_v, sem)}
        oh = {0: None, 1: None}
        for l in range(_NUM_LEVELS):
            p = l & 1
            frac_v, idx_v, pos_v, rows_v, sem = bufs[p]
            if l + 1 < _NUM_LEVELS:
                nfrac, nidx, npos, nrows, nsem = bufs[1 - p]
                hash_level(l + 1, nfrac, nidx, npos)
                gh[1 - p] = pltpu.async_copy(tbl_hbm.at[nidx], nrows, nsem)
            gh[p].wait()
            outb_v, sem_o = obufs[p]
            if oh[p] is not None:
                oh[p].wait()
            blend_level(frac_v, pos_v, rows_v, outb_v)
            oh[p] = pltpu.async_copy(
                outb_v, out_hbm.at[b, pl.ds(2 * l, 2), pl.ds(n0, _C)], sem_o)
        oh[0].wait()
        oh[1].wait()
        return carry

    lax.fori_loop(0, _CHUNKS, chunk_body, 0)


def kernel(coords, tables):
    cnorm = pl.pallas_call(
        _tc_normalize,
        out_shape=jax.ShapeDtypeStruct((_B, 3, _N), jnp.float32),
    )(coords)

    tv = tables.reshape(_NUM_LEVELS, 2**_LOG2 // 128, 128, _F).swapaxes(2, 3)
    ncell = (2**_LOG2 // 128) // _BT
    tbl128 = pl.pallas_call(
        _tc_interleave,
        grid=(_NUM_LEVELS, ncell),
        in_specs=[pl.BlockSpec((1, _BT, _F, 128), lambda l, c: (l, c, 0, 0))],
        out_specs=pl.BlockSpec((2 * _BT, 128), lambda l, c: (l * ncell + c, 0)),
        out_shape=jax.ShapeDtypeStruct(
            (_NUM_LEVELS * 2**_LOG2 * _F // 128, 128), jnp.float32),
    )(tv)
    tbl = tbl128.reshape(_NUM_LEVELS * 2**_LOG2 * _F // 8, 8)

    mesh = plsc.VectorSubcoreMesh(core_axis_name="c", subcore_axis_name="s")
    sc = functools.partial(
        pl.kernel,
        mesh=mesh,
        compiler_params=pltpu.CompilerParams(
            needs_layout_passes=False, use_tc_tiling_on_sc=False
        ),
        out_type=jax.ShapeDtypeStruct((_B, 2 * _NUM_LEVELS, _N), jnp.float32),
        scratch_types=[
            pltpu.VMEM((3, _C), jnp.float32),
            pltpu.VMEM((3, _C), jnp.float32),
            pltpu.VMEM((3, _C), jnp.float32),
            pltpu.VMEM((8 * _C,), jnp.int32),
            pltpu.VMEM((8 * _C,), jnp.int32),
            pltpu.VMEM((8 * _C,), jnp.int32),
            pltpu.VMEM((8 * _C,), jnp.int32),
            pltpu.VMEM((8 * _C, 8), jnp.float32),
            pltpu.VMEM((8 * _C, 8), jnp.float32),
            pltpu.VMEM((2, _C), jnp.float32),
            pltpu.VMEM((2, _C), jnp.float32),
            pltpu.SemaphoreType.DMA,
            pltpu.SemaphoreType.DMA,
            pltpu.SemaphoreType.DMA,
            pltpu.SemaphoreType.DMA,
        ],
    )(_sc_body)
    return sc(cnorm, tbl)


# speedup vs baseline: 29.2988x; 1.1149x over previous
import functools

import jax
import jax.numpy as jnp
import numpy as np
from jax import lax
from jax.experimental import pallas as pl
from jax.experimental.pallas import tpu as pltpu
from jax.experimental.pallas import tpu_sc as plsc

_NUM_LEVELS = 16
_BASE_RES = 16
_LOG2 = 21
_F = 2
_B = 4
_N = 65536

_P1 = int(np.uint32(2654435761).astype(np.int32))
_P2 = 805459861
_MASK = 2**_LOG2 - 1

_NC = 2
_NS = 16
_NW = _NC * _NS
_PTS = _B * _N
_PER_W = _PTS // _NW
_C = 512
_CHUNKS = _PER_W // _C
_GROUPS = _C // 16
_WPB = _N // _PER_W

_BT = 2048


def _tc_normalize(c_ref, o_ref):
    c = c_ref[...]
    cmin = jnp.min(c, axis=2, keepdims=True)
    cmax = jnp.max(c, axis=2, keepdims=True)
    o_ref[...] = (c - cmin) / (cmax - cmin + 1e-6)


def _tc_interleave(t_ref, o_ref):
    x = t_ref[0]
    x2 = jnp.concatenate([x[:, 0, :], x[:, 1, :]], axis=1)
    row = lax.broadcasted_iota(jnp.int32, (256, 128), 0)
    lane = lax.broadcasted_iota(jnp.int32, (256, 128), 1)
    cond_e = ((row < 64) & (lane == 2 * row)) | (
        (row >= 128) & (row < 192) & (lane == 2 * (row - 128) + 1))
    cond_o = ((row >= 64) & (row < 128) & (lane == 2 * (row - 64))) | (
        (row >= 192) & (lane == 2 * (row - 192) + 1))
    m_e = jnp.where(cond_e, 1.0, 0.0).astype(jnp.bfloat16)
    m_o = jnp.where(cond_o, 1.0, 0.0).astype(jnp.bfloat16)
    hi = x2.astype(jnp.bfloat16)
    r1 = x2 - hi.astype(jnp.float32)
    mid = r1.astype(jnp.bfloat16)
    lo = (r1 - mid.astype(jnp.float32)).astype(jnp.bfloat16)

    def pdot(a, m):
        return jax.lax.dot(a, m, preferred_element_type=jnp.float32)

    o_e = (pdot(hi, m_e) + pdot(mid, m_e)) + pdot(lo, m_e)
    o_o = (pdot(hi, m_o) + pdot(mid, m_o)) + pdot(lo, m_o)
    y = jnp.concatenate([o_e[:, None, :], o_o[:, None, :]], axis=1)
    o_ref[...] = y.reshape(2 * _BT, 128)


def _sc_body(cn_hbm, tbl_hbm, out_hbm, cn_v,
             frac_a, frac_b, idx_a, idx_b, pos_a, pos_b, rows_a, rows_b,
             outb_a, outb_b, sem_a, sem_b, sem_oa, sem_ob):
    wid = lax.axis_index("s") * _NC + lax.axis_index("c")
    b = wid // _WPB
    nbase = (wid % _WPB) * _PER_W
    iota = lax.broadcasted_iota(jnp.int32, (16,), 0)
    bufs = ((frac_a, idx_a, pos_a, rows_a, sem_a),
            (frac_b, idx_b, pos_b, rows_b, sem_b))
    obufs = ((outb_a, sem_oa), (outb_b, sem_ob))

    def hash_level(l, frac_v, idx_v, pos_v):
        scale = float(_BASE_RES * 2.0 ** l)
        lvl_off = l << (_LOG2 - 2)

        def hash_body(g, carry):
            p0 = g * 16
            xs = cn_v[0, pl.ds(p0, 16)] * scale
            ys = cn_v[1, pl.ds(p0, 16)] * scale
            zs = cn_v[2, pl.ds(p0, 16)] * scale
            xi = xs.astype(jnp.int32)
            yi = ys.astype(jnp.int32)
            zi = zs.astype(jnp.int32)
            frac_v[0, pl.ds(p0, 16)] = xs - xi.astype(jnp.float32)
            frac_v[1, pl.ds(p0, 16)] = ys - yi.astype(jnp.float32)
            frac_v[2, pl.ds(p0, 16)] = zs - zi.astype(jnp.float32)
            ax0 = xi
            ax1 = xi + 1
            by0 = yi * _P1
            by1 = by0 + _P1
            cz0 = zi * _P2
            cz1 = cz0 + _P2
            for k in range(8):
                a = ax1 if (k >> 2) & 1 else ax0
                bb = by1 if (k >> 1) & 1 else by0
                cc = cz1 if k & 1 else cz0
                h = (a + bb + cc) & _MASK
                idx_v[pl.ds(k * _C + p0, 16)] = (
                    lax.shift_right_logical(h, 2) + lvl_off)
                pos_v[pl.ds(k * _C + p0, 16)] = (h & 3) * 2
            return carry

        lax.fori_loop(0, _GROUPS, hash_body, 0)

    def blend_level(frac_v, pos_v, rows_v, outb_v):
        def blend_body(g, carry):
            p0 = g * 16
            fx = frac_v[0, pl.ds(p0, 16)]
            fy = frac_v[1, pl.ds(p0, 16)]
            fz = frac_v[2, pl.ds(p0, 16)]
            gx = 1.0 - fx
            gy = 1.0 - fy
            gz = 1.0 - fz
            t00 = gy * gz
            t01 = gy * fz
            t10 = fy * gz
            t11 = fy * fz
            ws = (gx * t00, gx * t01, gx * t10, gx * t11,
                  fx * t00, fx * t01, fx * t10, fx * t11)
            acc0 = jnp.zeros((16,), jnp.float32)
            acc1 = jnp.zeros((16,), jnp.float32)
            for k in range(8):
                rowid = k * _C + p0 + iota
                pos2 = pos_v[pl.ds(k * _C + p0, 16)]
                r0 = plsc.load_gather(rows_v, [rowid, pos2])
                r1 = plsc.load_gather(rows_v, [rowid, pos2 + 1])
                acc0 = acc0 + ws[k] * r0
                acc1 = acc1 + ws[k] * r1
            outb_v[0, pl.ds(p0, 16)] = acc0
            outb_v[1, pl.ds(p0, 16)] = acc1
            return carry

        lax.fori_loop(0, _GROUPS, blend_body, 0)

    def chunk_body(ci, carry):
        n0 = nbase + ci * _C
        pltpu.sync_copy(cn_hbm.at[b, :, pl.ds(n0, _C)], cn_v)

        frac_v, idx_v, pos_v, rows_v, sem = bufs[0]
        hash_level(0, frac_v, idx_v, pos_v)
        gh = {0: pltpu.async_copy(tbl_hbm.at[idx_v], rows_v, sem)}
        oh = {0: None, 1: None}
        for l in range(_NUM_LEVELS):
            p = l & 1
            frac_v, idx_v, pos_v, rows_v, sem = bufs[p]
            if l + 1 < _NUM_LEVELS:
                nfrac, nidx, npos, nrows, nsem = bufs[1 - p]
                hash_level(l + 1, nfrac, nidx, npos)
                gh[1 - p] = pltpu.async_copy(tbl_hbm.at[nidx], nrows, nsem)
            gh[p].wait()
            outb_v, sem_o = obufs[p]
            if oh[p] is not None:
                oh[p].wait()
            blend_level(frac_v, pos_v, rows_v, outb_v)
            oh[p] = pltpu.async_copy(
                outb_v, out_hbm.at[b, pl.ds(2 * l, 2), pl.ds(n0, _C)], sem_o)
        oh[0].wait()
        oh[1].wait()
        return carry

    lax.fori_loop(0, _CHUNKS, chunk_body, 0)


def kernel(coords, tables):
    cnorm = pl.pallas_call(
        _tc_normalize,
        out_shape=jax.ShapeDtypeStruct((_B, 3, _N), jnp.float32),
    )(coords)

    tv = tables.reshape(_NUM_LEVELS, 2**_LOG2 // 128, 128, _F).swapaxes(2, 3)
    ncell = (2**_LOG2 // 128) // _BT
    tbl128 = pl.pallas_call(
        _tc_interleave,
        grid=(_NUM_LEVELS, ncell),
        in_specs=[pl.BlockSpec((1, _BT, _F, 128), lambda l, c: (l, c, 0, 0))],
        out_specs=pl.BlockSpec((2 * _BT, 128), lambda l, c: (l * ncell + c, 0)),
        out_shape=jax.ShapeDtypeStruct(
            (_NUM_LEVELS * 2**_LOG2 * _F // 128, 128), jnp.float32),
    )(tv)
    tbl = tbl128.reshape(_NUM_LEVELS * 2**_LOG2 * _F // 8, 8)

    mesh = plsc.VectorSubcoreMesh(core_axis_name="c", subcore_axis_name="s")
    sc = functools.partial(
        pl.kernel,
        mesh=mesh,
        compiler_params=pltpu.CompilerParams(
            needs_layout_passes=False, use_tc_tiling_on_sc=False
        ),
        out_type=jax.ShapeDtypeStruct((_B, 2 * _NUM_LEVELS, _N), jnp.float32),
        scratch_types=[
            pltpu.VMEM((3, _C), jnp.float32),
            pltpu.VMEM((3, _C), jnp.float32),
            pltpu.VMEM((3, _C), jnp.float32),
            pltpu.VMEM((8 * _C,), jnp.int32),
            pltpu.VMEM((8 * _C,), jnp.int32),
            pltpu.VMEM((8 * _C,), jnp.int32),
            pltpu.VMEM((8 * _C,), jnp.int32),
            pltpu.VMEM((8 * _C, 8), jnp.float32),
            pltpu.VMEM((8 * _C, 8), jnp.float32),
            pltpu.VMEM((2, _C), jnp.float32),
            pltpu.VMEM((2, _C), jnp.float32),
            pltpu.SemaphoreType.DMA,
            pltpu.SemaphoreType.DMA,
            pltpu.SemaphoreType.DMA,
            pltpu.SemaphoreType.DMA,
        ],
    )(_sc_body)
    return sc(cnorm, tbl)
